# baseline (speedup 1.0000x reference)
import functools

import jax
import jax.numpy as jnp
from jax import lax
from jax.experimental import pallas as pl
from jax.experimental.pallas import tpu as pltpu
from jax.experimental.pallas import tpu_sc as plsc

_NB = 4
_SP = 64
_RP = 56


def _mesh_info():
    info = plsc.get_sparse_core_info()
    return info.num_cores, info.num_subcores


def _block_call(V, Bt, S):
    NC, NS = _mesh_info()
    bt_per_w = Bt // (NC * NS)
    n_chunks = bt_per_w // _NB
    mesh = plsc.VectorSubcoreMesh(core_axis_name="c", subcore_axis_name="s")

    @functools.partial(
        pl.kernel,
        mesh=mesh,
        out_type=jax.ShapeDtypeStruct((Bt, S, 128), jnp.float32),
        scratch_types=[
            pltpu.VMEM((_NB, _SP), jnp.int32),
            pltpu.VMEM((_NB, _RP, 128), jnp.float32),
            pltpu.SemaphoreType.DMA,
        ],
        compiler_params=pltpu.CompilerParams(needs_layout_passes=False),
    )
    def block_kernel(tb_hbm, idx_hbm, out_hbm, idx_v, buf_v, sem):
        wid = lax.axis_index("s") * NC + lax.axis_index("c")
        base_b = wid * bt_per_w

        def chunk(g, carry):
            bb = base_b + g * _NB
            for k in range(_NB):
                pltpu.sync_copy(idx_hbm.at[bb + k, :],
                                idx_v.at[k, pl.ds(0, S)])
            cps = [pltpu.async_copy(
                tb_hbm.at[idx_v.at[k, pl.ds(0, S)], :],
                buf_v.at[k, pl.ds(0, S), :], sem) for k in range(_NB)]
            for cp in cps:
                cp.wait()
            for k in range(_NB):
                pltpu.sync_copy(buf_v.at[k, pl.ds(0, S), :],
                                out_hbm.at[bb + k])
            return carry

        lax.fori_loop(0, n_chunks, chunk, 0)

    return block_kernel


def _rem_call(V, REM, Bt, S):
    NC, NS = _mesh_info()
    bt_per_w = Bt // (NC * NS)
    n_chunks = bt_per_w // _NB
    NG = S // 16
    TAIL = S - NG * 16
    mesh = plsc.VectorSubcoreMesh(core_axis_name="c", subcore_axis_name="s")

    @functools.partial(
        pl.kernel,
        mesh=mesh,
        out_type=jax.ShapeDtypeStruct((Bt, S, REM), jnp.float32),
        scratch_types=[
            pltpu.VMEM((_NB, _SP), jnp.int32),
            pltpu.VMEM((_NB, _RP, REM), jnp.float32),
            pltpu.SemaphoreType.DMA,
        ],
        compiler_params=pltpu.CompilerParams(needs_layout_passes=False),
    )
    def rem_kernel(tb_hbm, idx_hbm, out_hbm, idx_v, rem_v, sem):
        wid = lax.axis_index("s") * NC + lax.axis_index("c")
        base_b = wid * bt_per_w
        lane = lax.iota(jnp.int32, 16)

        def chunk(g, carry):
            bb = base_b + g * _NB
            for k in range(_NB):
                pltpu.sync_copy(idx_hbm.at[bb + k, :],
                                idx_v.at[k, pl.ds(0, S)])

            def row16(t, carry2):
                k = t // NG
                o = (t % NG) * 16
                iv = idx_v[k, pl.ds(o, 16)]
                for j in range(16):
                    r = jnp.max(jnp.where(lane == j, iv, 0))
                    pltpu.async_copy(
                        tb_hbm.at[pl.ds(r, 1), :],
                        rem_v.at[k, pl.ds(o + j, 1), :], sem)
                return carry2

            lax.fori_loop(0, _NB * NG, row16, 0)
            for k in range(_NB):
                iv = idx_v[k, pl.ds(NG * 16, 16)]
                for j in range(TAIL):
                    r = jnp.max(jnp.where(lane == j, iv, 0))
                    pltpu.async_copy(
                        tb_hbm.at[pl.ds(r, 1), :],
                        rem_v.at[k, pl.ds(NG * 16 + j, 1), :], sem)
            for k in range(_NB):
                pltpu.make_async_copy(
                    out_hbm.at[0],
                    rem_v.at[k, pl.ds(0, S), :], sem).wait()
            for k in range(_NB):
                pltpu.sync_copy(rem_v.at[k, pl.ds(0, S), :],
                                out_hbm.at[bb + k])
            return carry

        lax.fori_loop(0, n_chunks, chunk, 0)

    return rem_kernel


def kernel(table, indices):
    V, D = table.shape
    Bt, S = indices.shape
    idx = indices.astype(jnp.int32)
    t0 = table[:, 0:128]
    t1 = table[:, 128:256]
    t2 = table[:, 256:300]
    o0 = _block_call(V, Bt, S)(t0, idx)
    o1 = _block_call(V, Bt, S)(t1, idx)
    o2 = _rem_call(V, D - 256, Bt, S)(t2, idx)
    return jnp.concatenate([o0, o1, o2], axis=2)


# speedup vs baseline: 1.3957x; 1.3957x over previous
import functools

import jax
import jax.numpy as jnp
from jax import lax
from jax.experimental import pallas as pl
from jax.experimental.pallas import tpu as pltpu
from jax.experimental.pallas import tpu_sc as plsc


def _gather_call(V, D, Bt, S):
    info = plsc.get_sparse_core_info()
    NC, NS = info.num_cores, info.num_subcores
    NW = NC * NS
    bt_per_w = Bt // NW
    NB = 2
    n_chunks = bt_per_w // NB
    SP = 64
    RP = 56
    REM = D - 256
    NG = S // 16
    TAIL = S - NG * 16

    mesh = plsc.VectorSubcoreMesh(core_axis_name="c", subcore_axis_name="s")

    @functools.partial(
        pl.kernel,
        mesh=mesh,
        out_type=jax.ShapeDtypeStruct((Bt, S, D), jnp.float32),
        scratch_types=[
            pltpu.VMEM((NB, SP), jnp.int32),
            pltpu.VMEM((NB, RP, 128), jnp.float32),
            pltpu.VMEM((NB, RP, 128), jnp.float32),
            pltpu.VMEM((NB, RP, REM), jnp.float32),
            pltpu.VMEM((NB, SP), jnp.int32),
            pltpu.VMEM((NB, RP, 128), jnp.float32),
            pltpu.VMEM((NB, RP, 128), jnp.float32),
            pltpu.VMEM((NB, RP, REM), jnp.float32),
            pltpu.SemaphoreType.DMA,
            pltpu.SemaphoreType.DMA,
            pltpu.SemaphoreType.DMA,
            pltpu.SemaphoreType.DMA,
            pltpu.SemaphoreType.DMA,
            pltpu.SemaphoreType.DMA,
        ],
        compiler_params=pltpu.CompilerParams(needs_layout_passes=False),
    )
    def gather_kernel(table_hbm, idx_hbm, out_hbm,
                      idx_a, b0_a, b1_a, rem_a,
                      idx_b, b0_b, b1_b, rem_b,
                      gsem_a, rsem_a, wsem_a, gsem_b, rsem_b, wsem_b):
        wid = lax.axis_index("s") * NC + lax.axis_index("c")
        base_b = wid * bt_per_w
        lane = lax.iota(jnp.int32, 16)
        sets = (
            (idx_a, b0_a, b1_a, rem_a, gsem_a, rsem_a, wsem_a),
            (idx_b, b0_b, b1_b, rem_b, gsem_b, rsem_b, wsem_b),
        )

        def drain_writes(p):
            idx_v, b0_v, b1_v, rem_v, gsem, rsem, wsem = sets[p]
            for k in range(NB):
                pltpu.make_async_copy(
                    b0_v.at[k, pl.ds(0, S), :],
                    out_hbm.at[0, :, pl.ds(0, 128)], wsem).wait()
                pltpu.make_async_copy(
                    b1_v.at[k, pl.ds(0, S), :],
                    out_hbm.at[0, :, pl.ds(128, 128)], wsem).wait()
                pltpu.make_async_copy(
                    rem_v.at[k, pl.ds(0, S), :],
                    out_hbm.at[0, :, pl.ds(256, REM)], wsem).wait()

        def issue(g, p):
            idx_v, b0_v, b1_v, rem_v, gsem, rsem, wsem = sets[p]
            bb = base_b + g * NB
            for k in range(NB):
                pltpu.sync_copy(idx_hbm.at[bb + k, :],
                                idx_v.at[k, pl.ds(0, S)])
            for k in range(NB):
                pltpu.async_copy(
                    table_hbm.at[idx_v.at[k, pl.ds(0, S)], pl.ds(0, 128)],
                    b0_v.at[k, pl.ds(0, S), :], gsem)
                pltpu.async_copy(
                    table_hbm.at[idx_v.at[k, pl.ds(0, S)], pl.ds(128, 128)],
                    b1_v.at[k, pl.ds(0, S), :], gsem)

            def row16(t, carry2):
                k = t // NG
                o = (t % NG) * 16
                iv = idx_v[k, pl.ds(o, 16)]
                for j in range(16):
                    r = jnp.max(jnp.where(lane == j, iv, 0))
                    pltpu.async_copy(
                        table_hbm.at[pl.ds(r, 1), pl.ds(256, REM)],
                        rem_v.at[k, pl.ds(o + j, 1), :], rsem)
                return carry2

            lax.fori_loop(0, NB * NG, row16, 0)
            for k in range(NB):
                iv = idx_v[k, pl.ds(NG * 16, 16)]
                for j in range(TAIL):
                    r = jnp.max(jnp.where(lane == j, iv, 0))
                    pltpu.async_copy(
                        table_hbm.at[pl.ds(r, 1), pl.ds(256, REM)],
                        rem_v.at[k, pl.ds(NG * 16 + j, 1), :], rsem)

        def complete(g, p):
            idx_v, b0_v, b1_v, rem_v, gsem, rsem, wsem = sets[p]
            bb = base_b + g * NB
            for k in range(NB):
                pltpu.make_async_copy(
                    out_hbm.at[0, :, pl.ds(0, 128)],
                    b0_v.at[k, pl.ds(0, S), :], gsem).wait()
                pltpu.make_async_copy(
                    out_hbm.at[0, :, pl.ds(128, 128)],
                    b1_v.at[k, pl.ds(0, S), :], gsem).wait()
                pltpu.make_async_copy(
                    out_hbm.at[0, :, pl.ds(256, REM)],
                    rem_v.at[k, pl.ds(0, S), :], rsem).wait()
            for k in range(NB):
                pltpu.async_copy(b0_v.at[k, pl.ds(0, S), :],
                                 out_hbm.at[bb + k, :, pl.ds(0, 128)], wsem)
                pltpu.async_copy(b1_v.at[k, pl.ds(0, S), :],
                                 out_hbm.at[bb + k, :, pl.ds(128, 128)], wsem)
                pltpu.async_copy(rem_v.at[k, pl.ds(0, S), :],
                                 out_hbm.at[bb + k, :, pl.ds(256, REM)], wsem)

        issue(0, 0)

        def pair(u, carry):
            g0 = u * 2

            @pl.when(u >= 1)
            def _():
                drain_writes(1)

            issue(g0 + 1, 1)
            complete(g0, 0)

            @pl.when(g0 + 2 < n_chunks)
            def _():
                drain_writes(0)
                issue(g0 + 2, 0)

            complete(g0 + 1, 1)
            return carry

        lax.fori_loop(0, n_chunks // 2, pair, 0)
        drain_writes(0)
        drain_writes(1)

    return gather_kernel


def kernel(table, indices):
    V, D = table.shape
    Bt, S = indices.shape
    idx = indices.astype(jnp.int32)
    return _gather_call(V, D, Bt, S)(table, idx)


# speedup vs baseline: 1.4276x; 1.0228x over previous
import functools

import jax
import jax.numpy as jnp
from jax import lax
from jax.experimental import pallas as pl
from jax.experimental.pallas import tpu as pltpu
from jax.experimental.pallas import tpu_sc as plsc


def _gather_call(V, D, Bt, S):
    info = plsc.get_sparse_core_info()
    NC, NS = info.num_cores, info.num_subcores
    NW = NC * NS
    bt_per_w = Bt // NW
    NB = 2
    n_chunks = bt_per_w // NB
    SP = 64
    RP = 56
    REM = D - 256
    NG = S // 16
    TAIL = S - NG * 16

    mesh = plsc.VectorSubcoreMesh(core_axis_name="c", subcore_axis_name="s")

    @functools.partial(
        pl.kernel,
        mesh=mesh,
        out_type=jax.ShapeDtypeStruct((Bt, S, D), jnp.float32),
        scratch_types=[
            pltpu.VMEM((NB, SP), jnp.int32),
            pltpu.VMEM((NB, RP, 128), jnp.float32),
            pltpu.VMEM((NB, RP, 128), jnp.float32),
            pltpu.VMEM((NB, RP, REM), jnp.float32),
            pltpu.VMEM((NB, SP), jnp.int32),
            pltpu.VMEM((NB, RP, 128), jnp.float32),
            pltpu.VMEM((NB, RP, 128), jnp.float32),
            pltpu.VMEM((NB, RP, REM), jnp.float32),
            pltpu.SemaphoreType.DMA,
            pltpu.SemaphoreType.DMA,
            pltpu.SemaphoreType.DMA,
            pltpu.SemaphoreType.DMA,
            pltpu.SemaphoreType.DMA,
            pltpu.SemaphoreType.DMA,
            pltpu.SemaphoreType.DMA,
            pltpu.SemaphoreType.DMA,
        ],
        compiler_params=pltpu.CompilerParams(needs_layout_passes=False),
    )
    def gather_kernel(table_hbm, idx_hbm, out_hbm,
                      idx_a, b0_a, b1_a, rem_a,
                      idx_b, b0_b, b1_b, rem_b,
                      gsem_a, rsem_a, wsem_a, isem_a,
                      gsem_b, rsem_b, wsem_b, isem_b):
        wid = lax.axis_index("s") * NC + lax.axis_index("c")
        base_b = wid * bt_per_w
        lane = lax.iota(jnp.int32, 16)
        sets = (
            (idx_a, b0_a, b1_a, rem_a, gsem_a, rsem_a, wsem_a, isem_a),
            (idx_b, b0_b, b1_b, rem_b, gsem_b, rsem_b, wsem_b, isem_b),
        )

        def stage(g, p):
            idx_v, _, _, _, _, _, _, isem = sets[p]
            bb = base_b + g * NB
            for k in range(NB):
                pltpu.async_copy(idx_hbm.at[bb + k, :],
                                 idx_v.at[k, pl.ds(0, S)], isem)

        def drain_writes(p):
            idx_v, b0_v, b1_v, rem_v, gsem, rsem, wsem, isem = sets[p]
            for k in range(NB):
                pltpu.make_async_copy(
                    b0_v.at[k, pl.ds(0, S), :],
                    out_hbm.at[0, :, pl.ds(0, 128)], wsem).wait()
                pltpu.make_async_copy(
                    b1_v.at[k, pl.ds(0, S), :],
                    out_hbm.at[0, :, pl.ds(128, 128)], wsem).wait()
                pltpu.make_async_copy(
                    rem_v.at[k, pl.ds(0, S), :],
                    out_hbm.at[0, :, pl.ds(256, REM)], wsem).wait()

        def issue(g, p):
            idx_v, b0_v, b1_v, rem_v, gsem, rsem, wsem, isem = sets[p]
            bb = base_b + g * NB
            for k in range(NB):
                pltpu.make_async_copy(idx_hbm.at[0, :],
                                      idx_v.at[k, pl.ds(0, S)], isem).wait()
            for k in range(NB):
                pltpu.async_copy(
                    table_hbm.at[idx_v.at[k, pl.ds(0, S)], pl.ds(0, 128)],
                    b0_v.at[k, pl.ds(0, S), :], gsem)
                pltpu.async_copy(
                    table_hbm.at[idx_v.at[k, pl.ds(0, S)], pl.ds(128, 128)],
                    b1_v.at[k, pl.ds(0, S), :], gsem)

            def row16(t, carry2):
                k = t // NG
                o = (t % NG) * 16
                iv = idx_v[k, pl.ds(o, 16)]
                for j in range(16):
                    r = jnp.max(jnp.where(lane == j, iv, 0))
                    pltpu.async_copy(
                        table_hbm.at[pl.ds(r, 1), pl.ds(256, REM)],
                        rem_v.at[k, pl.ds(o + j, 1), :], rsem)
                return carry2

            lax.fori_loop(0, NB * NG, row16, 0)
            for k in range(NB):
                iv = idx_v[k, pl.ds(NG * 16, 16)]
                for j in range(TAIL):
                    r = jnp.max(jnp.where(lane == j, iv, 0))
                    pltpu.async_copy(
                        table_hbm.at[pl.ds(r, 1), pl.ds(256, REM)],
                        rem_v.at[k, pl.ds(NG * 16 + j, 1), :], rsem)

        def complete(g, p):
            idx_v, b0_v, b1_v, rem_v, gsem, rsem, wsem, isem = sets[p]
            bb = base_b + g * NB
            for k in range(NB):
                pltpu.make_async_copy(
                    out_hbm.at[0, :, pl.ds(0, 128)],
                    b0_v.at[k, pl.ds(0, S), :], gsem).wait()
                pltpu.make_async_copy(
                    out_hbm.at[0, :, pl.ds(128, 128)],
                    b1_v.at[k, pl.ds(0, S), :], gsem).wait()
                pltpu.make_async_copy(
                    out_hbm.at[0, :, pl.ds(256, REM)],
                    rem_v.at[k, pl.ds(0, S), :], rsem).wait()
            for k in range(NB):
                pltpu.async_copy(b0_v.at[k, pl.ds(0, S), :],
                                 out_hbm.at[bb + k, :, pl.ds(0, 128)], wsem)
                pltpu.async_copy(b1_v.at[k, pl.ds(0, S), :],
                                 out_hbm.at[bb + k, :, pl.ds(128, 128)], wsem)
                pltpu.async_copy(rem_v.at[k, pl.ds(0, S), :],
                                 out_hbm.at[bb + k, :, pl.ds(256, REM)], wsem)

        stage(0, 0)
        issue(0, 0)
        stage(1, 1)

        def pair(u, carry):
            g0 = u * 2

            @pl.when(u >= 1)
            def _():
                drain_writes(1)

            issue(g0 + 1, 1)
            complete(g0, 0)

            @pl.when(g0 + 2 < n_chunks)
            def _():
                stage(g0 + 2, 0)
                drain_writes(0)
                issue(g0 + 2, 0)

            complete(g0 + 1, 1)

            @pl.when(g0 + 3 < n_chunks)
            def _():
                stage(g0 + 3, 1)

            return carry

        lax.fori_loop(0, n_chunks // 2, pair, 0)
        drain_writes(0)
        drain_writes(1)

    return gather_kernel


def kernel(table, indices):
    V, D = table.shape
    Bt, S = indices.shape
    idx = indices.astype(jnp.int32)
    return _gather_call(V, D, Bt, S)(table, idx)
